# baseline (device time: 55503 ns/iter reference)
import jax
import jax.numpy as jnp
from jax import lax
from jax.experimental import pallas as pl
from jax.experimental.pallas import tpu as pltpu

EPS = 1e-6


def kernel(partial, gamma):
    p = partial.reshape(partial.shape[1], partial.shape[2])
    g = gamma.reshape(1, -1)
    m2, d = p.shape
    m = m2 // 2

    def body(p_ref, g_ref, out_ref, recv_buf, send_sem, recv_sem):
        my_x = lax.axis_index("x")
        my_y = lax.axis_index("y")
        my_z = lax.axis_index("z")
        peer_x = 1 - my_x

        barrier = pltpu.get_barrier_semaphore()
        pl.semaphore_signal(
            barrier, inc=1,
            device_id=(peer_x, my_y, my_z),
            device_id_type=pl.DeviceIdType.MESH,
        )
        pl.semaphore_wait(barrier, 1)

        rdma = pltpu.make_async_remote_copy(
            src_ref=p_ref.at[pl.ds(peer_x * m, m), :],
            dst_ref=recv_buf,
            send_sem=send_sem,
            recv_sem=recv_sem,
            device_id=(peer_x, my_y, my_z),
            device_id_type=pl.DeviceIdType.MESH,
        )
        rdma.start()
        rdma.wait()

        y = p_ref[pl.ds(my_x * m, m), :] + recv_buf[:, :]
        ms = jnp.mean(y * y, axis=-1, keepdims=True)
        out_ref[:, :] = y * lax.rsqrt(ms + EPS) * g_ref[0, :]

    return pl.pallas_call(
        body,
        out_shape=jax.ShapeDtypeStruct((m, d), jnp.float32),
        in_specs=[
            pl.BlockSpec(memory_space=pltpu.VMEM),
            pl.BlockSpec(memory_space=pltpu.VMEM),
        ],
        out_specs=pl.BlockSpec(memory_space=pltpu.VMEM),
        scratch_shapes=[
            pltpu.VMEM((m, d), jnp.float32),
            pltpu.SemaphoreType.DMA,
            pltpu.SemaphoreType.DMA,
        ],
        compiler_params=pltpu.CompilerParams(collective_id=0),
    )(p, g)


# device time: 37794 ns/iter; 1.4686x vs baseline; 1.4686x over previous
import jax
import jax.numpy as jnp
from jax import lax
from jax.experimental import pallas as pl
from jax.experimental.pallas import tpu as pltpu

EPS = 1e-6
N_CHUNK = 8


def kernel(partial, gamma):
    p = partial.reshape(partial.shape[1], partial.shape[2])
    g = gamma.reshape(1, -1)
    m2, d = p.shape
    m = m2 // 2
    half = m // 2
    ck = half // N_CHUNK

    def body(p_ref, g_ref, out_ref, recv_x, x_send, x_recv, z_send, z_recv):
        my_x = lax.axis_index("x")
        my_y = lax.axis_index("y")
        my_z = lax.axis_index("z")
        peer_x = 1 - my_x
        nbr_z = my_z ^ 1
        s = my_z % 2
        blk = s * half

        barrier = pltpu.get_barrier_semaphore()
        pl.semaphore_signal(
            barrier, inc=1,
            device_id=(peer_x, my_y, my_z),
            device_id_type=pl.DeviceIdType.MESH,
        )
        pl.semaphore_signal(
            barrier, inc=1,
            device_id=(my_x, my_y, nbr_z),
            device_id_type=pl.DeviceIdType.MESH,
        )
        pl.semaphore_wait(barrier, 2)

        x_rdmas = []
        for c in range(N_CHUNK):
            rdma = pltpu.make_async_remote_copy(
                src_ref=p_ref.at[pl.ds(peer_x * m + blk + c * ck, ck), :],
                dst_ref=recv_x.at[pl.ds(c * ck, ck), :],
                send_sem=x_send.at[c],
                recv_sem=x_recv.at[c],
                device_id=(peer_x, my_y, my_z),
                device_id_type=pl.DeviceIdType.MESH,
            )
            rdma.start()
            x_rdmas.append(rdma)

        z_rdmas = []
        for c in range(N_CHUNK):
            x_rdmas[c].wait_recv()
            y = p_ref[pl.ds(my_x * m + blk + c * ck, ck), :] + recv_x[
                pl.ds(c * ck, ck), :
            ]
            ms = jnp.mean(y * y, axis=-1, keepdims=True)
            out_ref[pl.ds(blk + c * ck, ck), :] = (
                y * lax.rsqrt(ms + EPS) * g_ref[0, :]
            )
            rdma = pltpu.make_async_remote_copy(
                src_ref=out_ref.at[pl.ds(blk + c * ck, ck), :],
                dst_ref=out_ref.at[pl.ds(blk + c * ck, ck), :],
                send_sem=z_send.at[c],
                recv_sem=z_recv.at[c],
                device_id=(my_x, my_y, nbr_z),
                device_id_type=pl.DeviceIdType.MESH,
            )
            rdma.start()
            z_rdmas.append(rdma)

        for c in range(N_CHUNK):
            x_rdmas[c].wait_send()
            z_rdmas[c].wait_send()
            z_rdmas[c].wait_recv()

    return pl.pallas_call(
        body,
        out_shape=jax.ShapeDtypeStruct((m, d), jnp.float32),
        in_specs=[
            pl.BlockSpec(memory_space=pltpu.VMEM),
            pl.BlockSpec(memory_space=pltpu.VMEM),
        ],
        out_specs=pl.BlockSpec(memory_space=pltpu.VMEM),
        scratch_shapes=[
            pltpu.VMEM((half, d), jnp.float32),
            pltpu.SemaphoreType.DMA((N_CHUNK,)),
            pltpu.SemaphoreType.DMA((N_CHUNK,)),
            pltpu.SemaphoreType.DMA((N_CHUNK,)),
            pltpu.SemaphoreType.DMA((N_CHUNK,)),
        ],
        compiler_params=pltpu.CompilerParams(collective_id=0),
    )(p, g)


# device time: 33247 ns/iter; 1.6694x vs baseline; 1.1368x over previous
import jax
import jax.numpy as jnp
from jax import lax
from jax.experimental import pallas as pl
from jax.experimental.pallas import tpu as pltpu

EPS = 1e-6
NC = 4


def kernel(partial, gamma):
    p = partial.reshape(partial.shape[1], partial.shape[2])
    g = gamma.reshape(1, -1)
    m2, d = p.shape
    m = m2 // 2
    qr = m // 4
    ck = qr // NC

    def body(
        p_ref, g_ref, out_ref, recv_x,
        x_send, x_recv,
        ydir_send, ydir_recv, zdir_send, zdir_recv,
        yfwd_send, yfwd_recv, zfwd_send, zfwd_recv,
    ):
        my_x = lax.axis_index("x")
        my_y = lax.axis_index("y")
        my_z = lax.axis_index("z")
        peer = (1 - my_x, my_y, my_z)
        nbr_y = (my_x, my_y ^ 1, my_z)
        nbr_z = (my_x, my_y, my_z ^ 1)
        a = my_y % 2
        b = my_z % 2
        q = 2 * a + b
        qy = q ^ 2
        qz = q ^ 1
        qd = q ^ 3

        def row(qi, c):
            return qi * qr + c * ck

        barrier = pltpu.get_barrier_semaphore()
        for dev in (peer, nbr_y, nbr_z):
            pl.semaphore_signal(
                barrier, inc=1, device_id=dev,
                device_id_type=pl.DeviceIdType.MESH,
            )
        pl.semaphore_wait(barrier, 3)

        x_rdmas = []
        for c in range(NC):
            rdma = pltpu.make_async_remote_copy(
                src_ref=p_ref.at[pl.ds((1 - my_x) * m + row(q, c), ck), :],
                dst_ref=recv_x.at[pl.ds(c * ck, ck), :],
                send_sem=x_send.at[c],
                recv_sem=x_recv.at[c],
                device_id=peer,
                device_id_type=pl.DeviceIdType.MESH,
            )
            rdma.start()
            x_rdmas.append(rdma)

        def block_rdma(qi, c, dev, send_sem, recv_sem):
            return pltpu.make_async_remote_copy(
                src_ref=out_ref.at[pl.ds(row(qi, c), ck), :],
                dst_ref=out_ref.at[pl.ds(row(qi, c), ck), :],
                send_sem=send_sem,
                recv_sem=recv_sem,
                device_id=dev,
                device_id_type=pl.DeviceIdType.MESH,
            )

        dir_rdmas = []
        for c in range(NC):
            x_rdmas[c].wait_recv()
            y = p_ref[pl.ds(my_x * m + row(q, c), ck), :] + recv_x[
                pl.ds(c * ck, ck), :
            ]
            ms = jnp.mean(y * y, axis=-1, keepdims=True)
            out_ref[pl.ds(row(q, c), ck), :] = (
                y * lax.rsqrt(ms + EPS) * g_ref[0, :]
            )
            ry = block_rdma(q, c, nbr_y, ydir_send.at[c], ydir_recv.at[c])
            ry.start()
            rz = block_rdma(q, c, nbr_z, zdir_send.at[c], zdir_recv.at[c])
            rz.start()
            dir_rdmas.append((ry, rz))

        zdir_in = [
            block_rdma(qz, c, nbr_z, zdir_send.at[c], zdir_recv.at[c])
            for c in range(NC)
        ]
        ydir_in = [
            block_rdma(qy, c, nbr_y, ydir_send.at[c], ydir_recv.at[c])
            for c in range(NC)
        ]
        fwd_rdmas = []
        for c in (0, 1):
            zdir_in[c].wait_recv()
            r = block_rdma(qz, c, nbr_y, yfwd_send.at[c], yfwd_recv.at[c])
            r.start()
            fwd_rdmas.append(r)
        for c in (2, 3):
            ydir_in[c].wait_recv()
            r = block_rdma(qy, c, nbr_z, zfwd_send.at[c - 2],
                           zfwd_recv.at[c - 2])
            r.start()
            fwd_rdmas.append(r)

        for c in (0, 1):
            ydir_in[c].wait_recv()
        for c in (2, 3):
            zdir_in[c].wait_recv()
        for c in (0, 1):
            block_rdma(qd, c, nbr_y, yfwd_send.at[c], yfwd_recv.at[c]).wait_recv()
        for c in (2, 3):
            block_rdma(qd, c, nbr_z, zfwd_send.at[c - 2],
                       zfwd_recv.at[c - 2]).wait_recv()
        for c in range(NC):
            x_rdmas[c].wait_send()
            dir_rdmas[c][0].wait_send()
            dir_rdmas[c][1].wait_send()
        for r in fwd_rdmas:
            r.wait_send()

    n_sem = pltpu.SemaphoreType.DMA
    return pl.pallas_call(
        body,
        out_shape=jax.ShapeDtypeStruct((m, d), jnp.float32),
        in_specs=[
            pl.BlockSpec(memory_space=pltpu.VMEM),
            pl.BlockSpec(memory_space=pltpu.VMEM),
        ],
        out_specs=pl.BlockSpec(memory_space=pltpu.VMEM),
        scratch_shapes=[
            pltpu.VMEM((qr, d), jnp.float32),
            n_sem((NC,)), n_sem((NC,)),
            n_sem((NC,)), n_sem((NC,)), n_sem((NC,)), n_sem((NC,)),
            n_sem((2,)), n_sem((2,)), n_sem((2,)), n_sem((2,)),
        ],
        compiler_params=pltpu.CompilerParams(collective_id=0),
    )(p, g)


# device time: 32185 ns/iter; 1.7245x vs baseline; 1.0330x over previous
import jax
import jax.numpy as jnp
from jax import lax
from jax.experimental import pallas as pl
from jax.experimental.pallas import tpu as pltpu

EPS = 1e-6
NC = 4


def kernel(partial, gamma):
    g = gamma.reshape(1, -1)
    _, m2, d = partial.shape
    m = m2 // 2
    qr = m // 4
    ck = qr // NC

    def body(
        p_ref, g_ref, out_ref, recv_x, local_p,
        local_sem, x_send, x_recv,
        ydir_send, ydir_recv, zdir_send, zdir_recv,
        yfwd_send, yfwd_recv, zfwd_send, zfwd_recv,
    ):
        my_x = lax.axis_index("x")
        my_y = lax.axis_index("y")
        my_z = lax.axis_index("z")
        peer = (1 - my_x, my_y, my_z)
        nbr_y = (my_x, my_y ^ 1, my_z)
        nbr_z = (my_x, my_y, my_z ^ 1)
        a = my_y % 2
        b = my_z % 2
        q = 2 * a + b
        qy = q ^ 2
        qz = q ^ 1
        qd = q ^ 3

        def row(qi, c):
            return qi * qr + c * ck

        local_copies = []
        for c in range(NC):
            cp = pltpu.make_async_copy(
                p_ref.at[0, pl.ds(my_x * m + row(q, c), ck), :],
                local_p.at[pl.ds(c * ck, ck), :],
                local_sem.at[c],
            )
            cp.start()
            local_copies.append(cp)

        barrier = pltpu.get_barrier_semaphore()
        for dev in (peer, nbr_y, nbr_z):
            pl.semaphore_signal(
                barrier, inc=1, device_id=dev,
                device_id_type=pl.DeviceIdType.MESH,
            )
        pl.semaphore_wait(barrier, 3)

        x_rdmas = []
        for c in range(NC):
            rdma = pltpu.make_async_remote_copy(
                src_ref=p_ref.at[0, pl.ds((1 - my_x) * m + row(q, c), ck), :],
                dst_ref=recv_x.at[pl.ds(c * ck, ck), :],
                send_sem=x_send.at[c],
                recv_sem=x_recv.at[c],
                device_id=peer,
                device_id_type=pl.DeviceIdType.MESH,
            )
            rdma.start()
            x_rdmas.append(rdma)

        def block_rdma(qi, c, dev, send_sem, recv_sem):
            return pltpu.make_async_remote_copy(
                src_ref=out_ref.at[pl.ds(row(qi, c), ck), :],
                dst_ref=out_ref.at[pl.ds(row(qi, c), ck), :],
                send_sem=send_sem,
                recv_sem=recv_sem,
                device_id=dev,
                device_id_type=pl.DeviceIdType.MESH,
            )

        dir_rdmas = []
        for c in range(NC):
            local_copies[c].wait()
            x_rdmas[c].wait_recv()
            y = local_p[pl.ds(c * ck, ck), :] + recv_x[pl.ds(c * ck, ck), :]
            ms = jnp.mean(y * y, axis=-1, keepdims=True)
            out_ref[pl.ds(row(q, c), ck), :] = (
                y * lax.rsqrt(ms + EPS) * g_ref[0, :]
            )
            ry = block_rdma(q, c, nbr_y, ydir_send.at[c], ydir_recv.at[c])
            ry.start()
            rz = block_rdma(q, c, nbr_z, zdir_send.at[c], zdir_recv.at[c])
            rz.start()
            dir_rdmas.append((ry, rz))

        zdir_in = [
            block_rdma(qz, c, nbr_z, zdir_send.at[c], zdir_recv.at[c])
            for c in range(NC)
        ]
        ydir_in = [
            block_rdma(qy, c, nbr_y, ydir_send.at[c], ydir_recv.at[c])
            for c in range(NC)
        ]
        fwd_rdmas = []
        for c in (0, 1):
            zdir_in[c].wait_recv()
            r = block_rdma(qz, c, nbr_y, yfwd_send.at[c], yfwd_recv.at[c])
            r.start()
            fwd_rdmas.append(r)
        for c in (2, 3):
            ydir_in[c].wait_recv()
            r = block_rdma(qy, c, nbr_z, zfwd_send.at[c - 2],
                           zfwd_recv.at[c - 2])
            r.start()
            fwd_rdmas.append(r)

        for c in (0, 1):
            ydir_in[c].wait_recv()
        for c in (2, 3):
            zdir_in[c].wait_recv()
        for c in (0, 1):
            block_rdma(qd, c, nbr_y, yfwd_send.at[c], yfwd_recv.at[c]).wait_recv()
        for c in (2, 3):
            block_rdma(qd, c, nbr_z, zfwd_send.at[c - 2],
                       zfwd_recv.at[c - 2]).wait_recv()
        for c in range(NC):
            x_rdmas[c].wait_send()
            dir_rdmas[c][0].wait_send()
            dir_rdmas[c][1].wait_send()
        for r in fwd_rdmas:
            r.wait_send()

    n_sem = pltpu.SemaphoreType.DMA
    return pl.pallas_call(
        body,
        out_shape=jax.ShapeDtypeStruct((m, d), jnp.float32),
        in_specs=[
            pl.BlockSpec(memory_space=pl.ANY),
            pl.BlockSpec(memory_space=pltpu.VMEM),
        ],
        out_specs=pl.BlockSpec(memory_space=pltpu.VMEM),
        scratch_shapes=[
            pltpu.VMEM((qr, d), jnp.float32),
            pltpu.VMEM((qr, d), jnp.float32),
            n_sem((NC,)), n_sem((NC,)), n_sem((NC,)),
            n_sem((NC,)), n_sem((NC,)), n_sem((NC,)), n_sem((NC,)),
            n_sem((2,)), n_sem((2,)), n_sem((2,)), n_sem((2,)),
        ],
        compiler_params=pltpu.CompilerParams(collective_id=0),
    )(partial, g)
